# baseline (speedup 1.0000x reference)
import functools

import jax
import jax.numpy as jnp
from jax import lax
from jax.experimental import pallas as pl
from jax.experimental.pallas import tpu as pltpu
from jax.experimental.pallas import tpu_sc as plsc

N_NODES = 2097152
N_GRAPHS = 16384
N_ELEM = 100
NF = 6
MF = 2

NC, NS, L = 2, 16, 16
NW = NC * NS
NT = N_NODES // NW
CH = 2048
NCHUNK = NT // CH
GT = N_GRAPHS // NW

_mesh = plsc.VectorSubcoreMesh(
    core_axis_name="c", subcore_axis_name="s", num_cores=NC, num_subcores=NS
)


@functools.partial(
    pl.kernel,
    out_type=(
        jax.ShapeDtypeStruct((NF, N_NODES), jnp.float32),
        jax.ShapeDtypeStruct((N_GRAPHS * MF,), jnp.float32),
    ),
    mesh=_mesh,
    compiler_params=pltpu.CompilerParams(needs_layout_passes=False, use_tc_tiling_on_sc=True),
    scratch_types=[
        pltpu.VMEM((NT,), jnp.int32),
        pltpu.VMEM((NF * CH,), jnp.float32),
        pltpu.VMEM((NF * CH,), jnp.float32),
        pltpu.VMEM((NF * 112,), jnp.float32),
        pltpu.VMEM((L,), jnp.float32),
        pltpu.VMEM((L,), jnp.float32),
        pltpu.VMEM((GT,), jnp.float32),
        pltpu.VMEM((GT,), jnp.float32),
        pltpu.VMEM((GT * MF,), jnp.float32),
        pltpu.SemaphoreType.DMA,
        pltpu.SemaphoreType.DMA,
    ],
)
def _featurize(x_hbm, na_hbm, rad_hbm, tab_hbm, nstd_hbm, gstd_hbm,
               node_out, mol_out,
               x_v, out_a, out_b, tab_v, nstd_v, gstd_v,
               na_v, rad_v, molo_v, sem_a, sem_b):
    wid = lax.axis_index("s") * NC + lax.axis_index("c")
    nbase = wid * NT
    gbase = wid * GT

    pltpu.sync_copy(x_hbm.at[pl.ds(nbase, NT)], x_v)
    pltpu.sync_copy(tab_hbm, tab_v)
    pltpu.sync_copy(nstd_hbm, nstd_v.at[pl.ds(0, NF * 2)])
    pltpu.sync_copy(gstd_hbm, gstd_v.at[pl.ds(0, MF * 2)])
    pltpu.sync_copy(na_hbm.at[pl.ds(gbase, GT)], na_v)
    pltpu.sync_copy(rad_hbm.at[pl.ds(gbase, GT)], rad_v)

    iota = lax.iota(jnp.int32, L)
    i6 = iota * NF
    i2 = iota * MF

    nv = nstd_v[pl.ds(0, L)]
    gv = gstd_v[pl.ds(0, L)]
    rnv = 1.0 / nv
    rgv = 1.0 / gv
    means = [jnp.full((L,), nv[2 * j]) for j in range(NF)]
    rstds = [jnp.full((L,), rnv[2 * j + 1]) for j in range(NF)]

    m_na = jnp.full((L,), gv[0])
    rs_na = jnp.full((L,), rgv[1])
    m_r = jnp.full((L,), gv[2])
    rs_r = jnp.full((L,), rgv[3])

    for j in range(NF):
        for g in range(7):
            sl = pl.ds(j * 112 + g * L, L)
            tab_v[sl] = (tab_v[sl] - means[j]) * rstds[j]

    def mol_body(g, carry):
        na = (na_v[pl.ds(g * L, L)] - m_na) * rs_na
        rd = (rad_v[pl.ds(g * L, L)] - m_r) * rs_r
        molo_v[pl.ds(g * L, L)] = na
        molo_v[pl.ds(GT + g * L, L)] = rd
        return carry

    lax.fori_loop(0, GT // L, mol_body, 0)
    pltpu.sync_copy(molo_v.at[pl.ds(0, GT)], mol_out.at[pl.ds(gbase, GT)])
    pltpu.sync_copy(molo_v.at[pl.ds(GT, GT)],
                    mol_out.at[pl.ds(N_GRAPHS + gbase, GT)])

    def run_chunk(ch, buf):
        @plsc.parallel_loop(0, CH, step=L, unroll=8)
        def _(i):
            xv = x_v[pl.ds(ch * CH + i, L)]
            for j in range(NF):
                buf[pl.ds(j * CH + i, L)] = plsc.load_gather(
                    tab_v, [xv + 112 * j] if j else [xv])

    bufs = (out_a, out_b)
    sems = (sem_a, sem_b)
    pending = [None, None]
    for ch in range(NCHUNK):
        b = ch % 2
        if pending[b] is not None:
            for d in pending[b]:
                d.wait()
        run_chunk(ch, bufs[b])
        col0 = nbase + ch * CH
        pending[b] = [
            pltpu.async_copy(bufs[b].at[pl.ds(j * CH, CH)],
                             node_out.at[j, pl.ds(col0, CH)],
                             sems[b])
            for j in range(NF)
        ]
    for p in pending:
        for d in p:
            d.wait()


def kernel(x, num_atoms, radius, atom_properties_tensor,
           node_standardization_tensor, graph_standardization_tensor):
    node_cm, mol_flat = _featurize(
        x, num_atoms, radius,
        jnp.pad(atom_properties_tensor.T, ((0, 0), (0, 112 - N_ELEM))).reshape(-1),
        node_standardization_tensor.reshape(-1),
        graph_standardization_tensor.reshape(-1))
    return (node_cm.T,
            mol_flat.reshape(MF, N_GRAPHS).T)


# speedup vs baseline: 1.0593x; 1.0593x over previous
import functools

import jax
import jax.numpy as jnp
from jax import lax
from jax.experimental import pallas as pl
from jax.experimental.pallas import tpu as pltpu
from jax.experimental.pallas import tpu_sc as plsc

N_NODES = 2097152
N_GRAPHS = 16384
N_ELEM = 100
NF = 6
MF = 2

NC, NS, L = 2, 16, 16
NW = NC * NS
NT = N_NODES // NW
CH = 4096
NCHUNK = NT // CH
GT = N_GRAPHS // NW

_mesh = plsc.VectorSubcoreMesh(
    core_axis_name="c", subcore_axis_name="s", num_cores=NC, num_subcores=NS
)


@functools.partial(
    pl.kernel,
    out_type=(
        jax.ShapeDtypeStruct((NF, N_NODES), jnp.float32),
        jax.ShapeDtypeStruct((N_GRAPHS * MF,), jnp.float32),
    ),
    mesh=_mesh,
    compiler_params=pltpu.CompilerParams(needs_layout_passes=False, use_tc_tiling_on_sc=True),
    scratch_types=[
        pltpu.VMEM((CH,), jnp.int32),
        pltpu.VMEM((CH,), jnp.int32),
        pltpu.VMEM((NF, CH), jnp.float32),
        pltpu.VMEM((NF, CH), jnp.float32),
        pltpu.VMEM((NF * 112,), jnp.float32),
        pltpu.VMEM((L,), jnp.float32),
        pltpu.VMEM((L,), jnp.float32),
        pltpu.VMEM((GT,), jnp.float32),
        pltpu.VMEM((GT,), jnp.float32),
        pltpu.VMEM((GT * MF,), jnp.float32),
        pltpu.SemaphoreType.DMA,
        pltpu.SemaphoreType.DMA,
        pltpu.SemaphoreType.DMA,
        pltpu.SemaphoreType.DMA,
    ],
)
def _featurize(x_hbm, na_hbm, rad_hbm, tab_hbm, nstd_hbm, gstd_hbm,
               node_out, mol_out,
               x_a, x_b, out_a, out_b, tab_v, nstd_v, gstd_v,
               na_v, rad_v, molo_v, sem_a, sem_b, sx_a, sx_b):
    wid = lax.axis_index("s") * NC + lax.axis_index("c")
    nbase = wid * NT
    gbase = wid * GT

    pltpu.sync_copy(tab_hbm, tab_v)
    pltpu.sync_copy(nstd_hbm, nstd_v.at[pl.ds(0, NF * 2)])
    pltpu.sync_copy(gstd_hbm, gstd_v.at[pl.ds(0, MF * 2)])
    pltpu.sync_copy(na_hbm.at[pl.ds(gbase, GT)], na_v)
    pltpu.sync_copy(rad_hbm.at[pl.ds(gbase, GT)], rad_v)

    iota = lax.iota(jnp.int32, L)
    i6 = iota * NF
    i2 = iota * MF

    nv = nstd_v[pl.ds(0, L)]
    gv = gstd_v[pl.ds(0, L)]
    rnv = 1.0 / nv
    rgv = 1.0 / gv
    means = [jnp.full((L,), nv[2 * j]) for j in range(NF)]
    rstds = [jnp.full((L,), rnv[2 * j + 1]) for j in range(NF)]

    m_na = jnp.full((L,), gv[0])
    rs_na = jnp.full((L,), rgv[1])
    m_r = jnp.full((L,), gv[2])
    rs_r = jnp.full((L,), rgv[3])

    for j in range(NF):
        for g in range(7):
            sl = pl.ds(j * 112 + g * L, L)
            tab_v[sl] = (tab_v[sl] - means[j]) * rstds[j]

    def mol_body(g, carry):
        na = (na_v[pl.ds(g * L, L)] - m_na) * rs_na
        rd = (rad_v[pl.ds(g * L, L)] - m_r) * rs_r
        molo_v[pl.ds(g * L, L)] = na
        molo_v[pl.ds(GT + g * L, L)] = rd
        return carry

    lax.fori_loop(0, GT // L, mol_body, 0)
    pltpu.sync_copy(molo_v.at[pl.ds(0, GT)], mol_out.at[pl.ds(gbase, GT)])
    pltpu.sync_copy(molo_v.at[pl.ds(GT, GT)],
                    mol_out.at[pl.ds(N_GRAPHS + gbase, GT)])

    def run_chunk(xbuf, buf):
        @plsc.parallel_loop(0, CH, step=L, unroll=8)
        def _(i):
            xv = xbuf[pl.ds(i, L)]
            for j in range(NF):
                buf[j, pl.ds(i, L)] = plsc.load_gather(
                    tab_v, [xv + 112 * j] if j else [xv])

    bufs = (out_a, out_b)
    sems = (sem_a, sem_b)
    xbufs = (x_a, x_b)
    xsems = (sx_a, sx_b)
    pending = [None, None]
    xpend = [None, None]
    xpend[0] = pltpu.async_copy(x_hbm.at[pl.ds(nbase, CH)], x_a, sx_a)
    for ch in range(NCHUNK):
        b = ch % 2
        if ch + 1 < NCHUNK:
            nb2 = (ch + 1) % 2
            xpend[nb2] = pltpu.async_copy(
                x_hbm.at[pl.ds(nbase + (ch + 1) * CH, CH)],
                xbufs[nb2], xsems[nb2])
        xpend[b].wait()
        if pending[b] is not None:
            for d in pending[b]:
                d.wait()
        run_chunk(xbufs[b], bufs[b])
        col0 = nbase + ch * CH
        pending[b] = [
            pltpu.async_copy(bufs[b],
                             node_out.at[:, pl.ds(col0, CH)],
                             sems[b])
        ]
    for p in pending:
        for d in p:
            d.wait()


def kernel(x, num_atoms, radius, atom_properties_tensor,
           node_standardization_tensor, graph_standardization_tensor):
    node_cm, mol_flat = _featurize(
        x, num_atoms, radius,
        jnp.pad(atom_properties_tensor.T, ((0, 0), (0, 112 - N_ELEM))).reshape(-1),
        node_standardization_tensor.reshape(-1),
        graph_standardization_tensor.reshape(-1))
    return (node_cm.T,
            mol_flat.reshape(MF, N_GRAPHS).T)


# speedup vs baseline: 1.0708x; 1.0109x over previous
import functools

import jax
import jax.numpy as jnp
from jax import lax
from jax.experimental import pallas as pl
from jax.experimental.pallas import tpu as pltpu
from jax.experimental.pallas import tpu_sc as plsc

N_NODES = 2097152
N_GRAPHS = 16384
N_ELEM = 100
NF = 6
MF = 2

NC, NS, L = 2, 16, 16
NW = NC * NS
NT = N_NODES // NW
CH = 4096
NCHUNK = NT // CH
GT = N_GRAPHS // NW

_mesh = plsc.VectorSubcoreMesh(
    core_axis_name="c", subcore_axis_name="s", num_cores=NC, num_subcores=NS
)


@functools.partial(
    pl.kernel,
    out_type=(
        jax.ShapeDtypeStruct((NF, N_NODES), jnp.float32),
        jax.ShapeDtypeStruct((N_GRAPHS * MF,), jnp.float32),
    ),
    mesh=_mesh,
    compiler_params=pltpu.CompilerParams(needs_layout_passes=False, use_tc_tiling_on_sc=True),
    scratch_types=[
        pltpu.VMEM((CH,), jnp.int32),
        pltpu.VMEM((CH,), jnp.int32),
        pltpu.VMEM((NF, CH), jnp.float32),
        pltpu.VMEM((NF, CH), jnp.float32),
        pltpu.VMEM((NF * 112,), jnp.float32),
        pltpu.VMEM((L,), jnp.float32),
        pltpu.VMEM((L,), jnp.float32),
        pltpu.VMEM((GT,), jnp.float32),
        pltpu.VMEM((GT,), jnp.float32),
        pltpu.VMEM((GT * MF,), jnp.float32),
        pltpu.SemaphoreType.DMA,
        pltpu.SemaphoreType.DMA,
        pltpu.SemaphoreType.DMA,
        pltpu.SemaphoreType.DMA,
    ],
)
def _featurize(x_hbm, na_hbm, rad_hbm, tab_hbm, nstd_hbm, gstd_hbm,
               node_out, mol_out,
               x_a, x_b, out_a, out_b, tab_v, nstd_v, gstd_v,
               na_v, rad_v, molo_v, sem_a, sem_b, sx_a, sx_b):
    wid = lax.axis_index("s") * NC + lax.axis_index("c")
    nbase = wid * NT
    gbase = wid * GT

    pltpu.sync_copy(tab_hbm, tab_v)
    pltpu.sync_copy(nstd_hbm, nstd_v.at[pl.ds(0, NF * 2)])
    pltpu.sync_copy(gstd_hbm, gstd_v.at[pl.ds(0, MF * 2)])
    pltpu.sync_copy(na_hbm.at[pl.ds(gbase, GT)], na_v)
    pltpu.sync_copy(rad_hbm.at[pl.ds(gbase, GT)], rad_v)

    iota = lax.iota(jnp.int32, L)
    i6 = iota * NF
    i2 = iota * MF

    nv = nstd_v[pl.ds(0, L)]
    gv = gstd_v[pl.ds(0, L)]
    rnv = 1.0 / nv
    rgv = 1.0 / gv
    means = [jnp.full((L,), nv[2 * j]) for j in range(NF)]
    rstds = [jnp.full((L,), rnv[2 * j + 1]) for j in range(NF)]

    m_na = jnp.full((L,), gv[0])
    rs_na = jnp.full((L,), rgv[1])
    m_r = jnp.full((L,), gv[2])
    rs_r = jnp.full((L,), rgv[3])

    for j in range(NF):
        for g in range(7):
            sl = pl.ds(j * 112 + g * L, L)
            tab_v[sl] = (tab_v[sl] - means[j]) * rstds[j]

    def mol_body(g, carry):
        na = (na_v[pl.ds(g * L, L)] - m_na) * rs_na
        rd = (rad_v[pl.ds(g * L, L)] - m_r) * rs_r
        molo_v[pl.ds(g * L, L)] = na
        molo_v[pl.ds(GT + g * L, L)] = rd
        return carry

    lax.fori_loop(0, GT // L, mol_body, 0)
    pltpu.sync_copy(molo_v.at[pl.ds(0, GT)], mol_out.at[pl.ds(gbase, GT)])
    pltpu.sync_copy(molo_v.at[pl.ds(GT, GT)],
                    mol_out.at[pl.ds(N_GRAPHS + gbase, GT)])

    def run_chunk(xbuf, buf):
        @plsc.parallel_loop(0, CH, step=L, unroll=4)
        def _(i):
            xv = xbuf[pl.ds(i, L)]
            for j in range(NF):
                buf[j, pl.ds(i, L)] = plsc.load_gather(
                    tab_v, [xv + 112 * j] if j else [xv])

    bufs = (out_a, out_b)
    sems = (sem_a, sem_b)
    xbufs = (x_a, x_b)
    xsems = (sx_a, sx_b)
    pending = [None, None]
    xpend = [None, None]
    xpend[0] = pltpu.async_copy(x_hbm.at[pl.ds(nbase, CH)], x_a, sx_a)
    for ch in range(NCHUNK):
        b = ch % 2
        if ch + 1 < NCHUNK:
            nb2 = (ch + 1) % 2
            xpend[nb2] = pltpu.async_copy(
                x_hbm.at[pl.ds(nbase + (ch + 1) * CH, CH)],
                xbufs[nb2], xsems[nb2])
        xpend[b].wait()
        if pending[b] is not None:
            for d in pending[b]:
                d.wait()
        run_chunk(xbufs[b], bufs[b])
        col0 = nbase + ch * CH
        pending[b] = [
            pltpu.async_copy(bufs[b],
                             node_out.at[:, pl.ds(col0, CH)],
                             sems[b])
        ]
    for p in pending:
        for d in p:
            d.wait()


def kernel(x, num_atoms, radius, atom_properties_tensor,
           node_standardization_tensor, graph_standardization_tensor):
    node_cm, mol_flat = _featurize(
        x, num_atoms, radius,
        jnp.pad(atom_properties_tensor.T, ((0, 0), (0, 112 - N_ELEM))).reshape(-1),
        node_standardization_tensor.reshape(-1),
        graph_standardization_tensor.reshape(-1))
    return (node_cm.T,
            mol_flat.reshape(MF, N_GRAPHS).T)
